# baseline (speedup 1.0000x reference)
import functools

import jax
import jax.numpy as jnp
from jax import lax
from jax.experimental import pallas as pl
from jax.experimental.pallas import tpu as pltpu, tpu_sc as plsc

NUM_CONCEPTS = 1000000
B_DIM = 16384
J_DIM = 26
DIM = 32
NW = 32
BH = 128
BH_W = BH // NW
ROWS_W = BH_W * 128


def _gather5d(idx_cols, table):
    mesh = plsc.VectorSubcoreMesh(core_axis_name="c", subcore_axis_name="s")

    @functools.partial(
        pl.kernel,
        mesh=mesh,
        out_type=jax.ShapeDtypeStruct((J_DIM, 4, BH, 8, 128), jnp.float32),
        scratch_types=[
            pltpu.VMEM((J_DIM, BH_W, 128), jnp.int32),
            pltpu.VMEM((2, ROWS_W, DIM), jnp.float32),
            pltpu.VMEM((2, 4, BH_W, 8, 128), jnp.float32),
            pltpu.SemaphoreType.DMA,
            pltpu.SemaphoreType.DMA,
        ],
        compiler_params=pltpu.CompilerParams(
            use_tc_tiling_on_sc=False, needs_layout_passes=False
        ),
    )
    def k(idx_hbm, tab_hbm, out_hbm, idx_v, rows_v, tb_v, sem_in, sem_out):
        wid = lax.axis_index("s") * 2 + lax.axis_index("c")
        pltpu.sync_copy(idx_hbm.at[pl.ds(0, J_DIM), pl.ds(wid * BH_W, BH_W)], idx_v)
        iota = lax.iota(jnp.int32, 16)

        def fire(j, par):
            for b in range(BH_W):
                pltpu.make_async_copy(
                    tab_hbm.at[idx_v.at[j, b]],
                    rows_v.at[par, pl.ds(b * 128, 128)],
                    sem_in,
                ).start()

        def wait_in(par):
            for b in range(BH_W):
                pltpu.make_async_copy(
                    tab_hbm.at[idx_v.at[0, b]],
                    rows_v.at[par, pl.ds(b * 128, 128)],
                    sem_in,
                ).wait()

        def transpose(par):
            rows = rows_v.at[par]
            for dh in range(4):
                for b in range(BH_W):
                    for dl in range(8):
                        col = jnp.full((16,), dh * 8 + dl, jnp.int32)
                        for c in range(8):
                            row = iota + (b * 128 + c * 16)
                            v = plsc.load_gather(rows, [row, col])
                            tb_v[par, dh, b, dl, pl.ds(c * 16, 16)] = v

        def start_out(j, par):
            for dh in range(4):
                pltpu.make_async_copy(
                    tb_v.at[par, dh],
                    out_hbm.at[j, dh, pl.ds(wid * BH_W, BH_W)],
                    sem_out,
                ).start()

        def drain_out(par):
            for dh in range(4):
                pltpu.make_async_copy(
                    tb_v.at[par, dh],
                    out_hbm.at[0, dh, pl.ds(wid * BH_W, BH_W)],
                    sem_out,
                ).wait()

        fire(0, 0)

        def body(jj, carry):
            j0 = 2 * jj
            j1 = j0 + 1
            fire(j1, 1)
            wait_in(0)

            @pl.when(jj > 0)
            def _():
                drain_out(0)

            transpose(0)
            start_out(j0, 0)

            @pl.when(jj < J_DIM // 2 - 1)
            def _():
                fire(j0 + 2, 0)

            wait_in(1)

            @pl.when(jj > 0)
            def _():
                drain_out(1)

            transpose(1)
            start_out(j1, 1)
            return carry

        lax.fori_loop(0, J_DIM // 2, body, 0)
        drain_out(0)
        drain_out(1)

    return k(idx_cols, table)


def kernel(idx, table):
    idx_cols = idx.T.reshape(J_DIM, BH, 128).astype(jnp.int32)
    out5d = _gather5d(idx_cols, table)
    return out5d.transpose(2, 4, 0, 1, 3).reshape(B_DIM, J_DIM, DIM)


# speedup vs baseline: 1.1637x; 1.1637x over previous
import functools

import jax
import jax.numpy as jnp
from jax import lax
from jax.experimental import pallas as pl
from jax.experimental.pallas import tpu as pltpu, tpu_sc as plsc

NUM_CONCEPTS = 1000000
B_DIM = 16384
J_DIM = 26
DIM = 32
NW = 32
BH = 128
BH_W = BH // NW
ROWS_W = BH_W * 128


def _gather5d(idx_cols, table):
    mesh = plsc.VectorSubcoreMesh(core_axis_name="c", subcore_axis_name="s")

    @functools.partial(
        pl.kernel,
        mesh=mesh,
        out_type=jax.ShapeDtypeStruct((J_DIM, 4, BH, 8, 128), jnp.float32),
        scratch_types=[
            pltpu.VMEM((J_DIM, BH_W, 128), jnp.int32),
            pltpu.VMEM((2, ROWS_W, DIM), jnp.float32),
            pltpu.VMEM((2, 4, BH_W, 8, 128), jnp.float32),
            pltpu.SemaphoreType.DMA,
            pltpu.SemaphoreType.DMA,
        ],
        compiler_params=pltpu.CompilerParams(
            use_tc_tiling_on_sc=False, needs_layout_passes=False
        ),
    )
    def k(idx_hbm, tab_hbm, out_hbm, idx_v, rows_v, tb_v, sem_in, sem_out):
        wid = lax.axis_index("s") * 2 + lax.axis_index("c")
        pltpu.sync_copy(idx_hbm.at[pl.ds(0, J_DIM), pl.ds(wid * BH_W, BH_W)], idx_v)
        iota = lax.iota(jnp.int32, 16)

        def fire(j, par):
            for b in range(BH_W):
                pltpu.make_async_copy(
                    tab_hbm.at[idx_v.at[j, b]],
                    rows_v.at[par, pl.ds(b * 128, 128)],
                    sem_in,
                ).start()

        def wait_in(par):
            for b in range(BH_W):
                pltpu.make_async_copy(
                    tab_hbm.at[idx_v.at[0, b]],
                    rows_v.at[par, pl.ds(b * 128, 128)],
                    sem_in,
                ).wait()

        def transpose(par):
            rows = rows_v.at[par]
            for dh in range(4):
                for b in range(BH_W):
                    for dl2 in range(4):
                        vs = []
                        for dl in (2 * dl2, 2 * dl2 + 1):
                            col = jnp.full((16,), dh * 8 + dl, jnp.int32)
                            for c in range(8):
                                row = iota + (b * 128 + c * 16)
                                vs.append(plsc.load_gather(rows, [row, col]))
                        i = 0
                        for dl in (2 * dl2, 2 * dl2 + 1):
                            for c in range(8):
                                tb_v[par, dh, b, dl, pl.ds(c * 16, 16)] = vs[i]
                                i += 1

        def start_out(j, par):
            for dh in range(4):
                pltpu.make_async_copy(
                    tb_v.at[par, dh],
                    out_hbm.at[j, dh, pl.ds(wid * BH_W, BH_W)],
                    sem_out,
                ).start()

        def drain_out(par):
            for dh in range(4):
                pltpu.make_async_copy(
                    tb_v.at[par, dh],
                    out_hbm.at[0, dh, pl.ds(wid * BH_W, BH_W)],
                    sem_out,
                ).wait()

        fire(0, 0)

        def body(jj, carry):
            j0 = 2 * jj
            j1 = j0 + 1
            fire(j1, 1)
            wait_in(0)

            @pl.when(jj > 0)
            def _():
                drain_out(0)

            transpose(0)
            start_out(j0, 0)

            @pl.when(jj < J_DIM // 2 - 1)
            def _():
                fire(j0 + 2, 0)

            wait_in(1)

            @pl.when(jj > 0)
            def _():
                drain_out(1)

            transpose(1)
            start_out(j1, 1)
            return carry

        lax.fori_loop(0, J_DIM // 2, body, 0)
        drain_out(0)
        drain_out(1)

    return k(idx_cols, table)


def kernel(idx, table):
    idx_cols = idx.T.reshape(J_DIM, BH, 128).astype(jnp.int32)
    out5d = _gather5d(idx_cols, table)
    return out5d.transpose(2, 4, 0, 1, 3).reshape(B_DIM, J_DIM, DIM)


# speedup vs baseline: 1.1801x; 1.0140x over previous
import functools

import jax
import jax.numpy as jnp
from jax import lax
from jax.experimental import pallas as pl
from jax.experimental.pallas import tpu as pltpu, tpu_sc as plsc

NUM_CONCEPTS = 1000000
B_DIM = 16384
J_DIM = 26
DIM = 32
NW = 32
BH = 128
BH_W = BH // NW
ROWS_W = BH_W * 128


def _gather5d(idx_cols, table):
    mesh = plsc.VectorSubcoreMesh(core_axis_name="c", subcore_axis_name="s")

    @functools.partial(
        pl.kernel,
        mesh=mesh,
        out_type=jax.ShapeDtypeStruct((J_DIM * 4 * BH * 8 * 128,), jnp.float32),
        scratch_types=[
            pltpu.VMEM((J_DIM, BH_W, 128), jnp.int32),
            pltpu.VMEM((2, ROWS_W, DIM), jnp.float32),
            pltpu.VMEM((2, 4 * BH_W * 8 * 128), jnp.float32),
            pltpu.SemaphoreType.DMA,
            pltpu.SemaphoreType.DMA,
        ],
        compiler_params=pltpu.CompilerParams(
            use_tc_tiling_on_sc=False, needs_layout_passes=False
        ),
    )
    def k(idx_hbm, tab_hbm, out_hbm, idx_v, rows_v, tb_v, sem_in, sem_out):
        wid = lax.axis_index("s") * 2 + lax.axis_index("c")
        pltpu.sync_copy(idx_hbm.at[pl.ds(0, J_DIM), pl.ds(wid * BH_W, BH_W)], idx_v)
        iota = lax.iota(jnp.int32, 16)

        def fire(j, par):
            for b in range(BH_W):
                pltpu.make_async_copy(
                    tab_hbm.at[idx_v.at[j, b]],
                    rows_v.at[par, pl.ds(b * 128, 128)],
                    sem_in,
                ).start()

        def wait_in(par):
            for b in range(BH_W):
                pltpu.make_async_copy(
                    tab_hbm.at[idx_v.at[0, b]],
                    rows_v.at[par, pl.ds(b * 128, 128)],
                    sem_in,
                ).wait()

        def transpose(par):
            rows = rows_v.at[par]

            def tbody(m, carry):
                dh = m // BH_W
                b = m % BH_W
                base = dh * 4096 + b * 1024
                for dl2 in range(4):
                    vs = []
                    for dl in (2 * dl2, 2 * dl2 + 1):
                        col = jnp.full((16,), dh * 8 + dl, jnp.int32)
                        for c in range(8):
                            row = iota + (b * 128 + c * 16)
                            vs.append(plsc.load_gather(rows, [row, col]))
                    i = 0
                    for dl in (2 * dl2, 2 * dl2 + 1):
                        for c in range(8):
                            tb_v[par, pl.ds(base + dl * 128 + c * 16, 16)] = vs[i]
                            i += 1
                return carry

            lax.fori_loop(0, 4 * BH_W, tbody, 0)

        def start_out(j, par):
            for dh in range(4):
                pltpu.make_async_copy(
                    tb_v.at[par, pl.ds(dh * 4096, 4096)],
                    out_hbm.at[pl.ds(((j * 4 + dh) * BH + wid * BH_W) * 1024, 4096)],
                    sem_out,
                ).start()

        def drain_out(par):
            for dh in range(4):
                pltpu.make_async_copy(
                    tb_v.at[par, pl.ds(dh * 4096, 4096)],
                    out_hbm.at[pl.ds(dh * 4096, 4096)],
                    sem_out,
                ).wait()

        fire(0, 0)

        def body(jj, carry):
            j0 = 2 * jj
            j1 = j0 + 1
            fire(j1, 1)
            wait_in(0)

            @pl.when(jj > 0)
            def _():
                drain_out(0)

            transpose(0)
            start_out(j0, 0)

            @pl.when(jj < J_DIM // 2 - 1)
            def _():
                fire(j0 + 2, 0)

            wait_in(1)

            @pl.when(jj > 0)
            def _():
                drain_out(1)

            transpose(1)
            start_out(j1, 1)
            return carry

        lax.fori_loop(0, J_DIM // 2, body, 0)
        drain_out(0)
        drain_out(1)

    return k(idx_cols, table)


def kernel(idx, table):
    idx_cols = idx.T.reshape(J_DIM, BH, 128).astype(jnp.int32)
    out1d = _gather5d(idx_cols, table)
    out5d = out1d.reshape(J_DIM, 4, BH, 8, 128)
    return out5d.transpose(2, 4, 0, 1, 3).reshape(B_DIM, J_DIM, DIM)


# speedup vs baseline: 1.5118x; 1.2811x over previous
import functools

import jax
import jax.numpy as jnp
from jax import lax
from jax.experimental import pallas as pl
from jax.experimental.pallas import tpu as pltpu, tpu_sc as plsc

NUM_CONCEPTS = 1000000
B_DIM = 16384
J_DIM = 26
DIM = 32
NW = 32
BH = 128
BH_W = BH // NW
ROWS_W = BH_W * 128


def _gather5d(idx_cols, table):
    mesh = plsc.VectorSubcoreMesh(core_axis_name="c", subcore_axis_name="s")

    @functools.partial(
        pl.kernel,
        mesh=mesh,
        out_type=jax.ShapeDtypeStruct((J_DIM, 4, BH * 8, 128), jnp.float32),
        scratch_types=[
            pltpu.VMEM((J_DIM, BH_W, 128), jnp.int32),
            pltpu.VMEM((2, ROWS_W, DIM), jnp.float32),
            pltpu.VMEM((2, 4, 40, 129), jnp.float32),
            pltpu.SemaphoreType.DMA,
            pltpu.SemaphoreType.DMA,
        ],
        compiler_params=pltpu.CompilerParams(
            use_tc_tiling_on_sc=False, needs_layout_passes=False
        ),
    )
    def k(idx_hbm, tab_hbm, out_hbm, idx_v, rows_v, tb_v, sem_in, sem_out):
        wid = lax.axis_index("s") * 2 + lax.axis_index("c")
        pltpu.sync_copy(idx_hbm.at[pl.ds(0, J_DIM), pl.ds(wid * BH_W, BH_W)], idx_v)
        iota = lax.iota(jnp.int32, 16)

        def fire(j, par):
            for b in range(BH_W):
                pltpu.make_async_copy(
                    tab_hbm.at[idx_v.at[j, b]],
                    rows_v.at[par, pl.ds(b * 128, 128)],
                    sem_in,
                ).start()

        def wait_in(par):
            for b in range(BH_W):
                pltpu.make_async_copy(
                    tab_hbm.at[idx_v.at[0, b]],
                    rows_v.at[par, pl.ds(b * 128, 128)],
                    sem_in,
                ).wait()

        dh_pat = lax.shift_right_logical(iota, 3)
        dl_pat = lax.bitwise_and(iota, 7)

        def transpose(par):
            parv = jnp.full((16,), par, jnp.int32)
            dh_hi = dh_pat + 2
            for b in range(BH_W):
                m_vec = dl_pat + (b * 8)

                def rbody(t, carry):
                    for u in range(8):
                        bl = t * 8 + u
                        r = b * 128 + bl
                        v1 = rows_v[par, r, pl.ds(0, 16)]
                        v2 = rows_v[par, r, pl.ds(16, 16)]
                        blv = jnp.full((16,), bl, jnp.int32)
                        plsc.store_scatter(tb_v, [parv, dh_pat, m_vec, blv], v1)
                        plsc.store_scatter(tb_v, [parv, dh_hi, m_vec, blv], v2)
                    return carry

                lax.fori_loop(0, 16, rbody, 0)

        def start_out(j, par):
            for dh in range(4):
                pltpu.make_async_copy(
                    tb_v.at[par, dh, pl.ds(0, 32), pl.ds(0, 128)],
                    out_hbm.at[j, dh, pl.ds(wid * 32, 32)],
                    sem_out,
                ).start()

        def drain_out(par):
            for dh in range(4):
                pltpu.make_async_copy(
                    tb_v.at[par, dh, pl.ds(0, 32), pl.ds(0, 128)],
                    out_hbm.at[0, dh, pl.ds(wid * 32, 32)],
                    sem_out,
                ).wait()

        fire(0, 0)

        def body(jj, carry):
            j0 = 2 * jj
            j1 = j0 + 1
            fire(j1, 1)
            wait_in(0)

            @pl.when(jj > 0)
            def _():
                drain_out(0)

            transpose(0)
            start_out(j0, 0)

            @pl.when(jj < J_DIM // 2 - 1)
            def _():
                fire(j0 + 2, 0)

            wait_in(1)

            @pl.when(jj > 0)
            def _():
                drain_out(1)

            transpose(1)
            start_out(j1, 1)
            return carry

        lax.fori_loop(0, J_DIM // 2, body, 0)
        drain_out(0)
        drain_out(1)

    return k(idx_cols, table)


def kernel(idx, table):
    idx_cols = idx.T.reshape(J_DIM, BH, 128).astype(jnp.int32)
    out4 = _gather5d(idx_cols, table)
    out5d = out4.reshape(J_DIM, 4, BH, 8, 128)
    return out5d.transpose(2, 4, 0, 1, 3).reshape(B_DIM, J_DIM, DIM)


# speedup vs baseline: 3.7184x; 2.4596x over previous
import functools

import jax
import jax.numpy as jnp
from jax import lax
from jax.experimental import pallas as pl
from jax.experimental.pallas import tpu as pltpu, tpu_sc as plsc

NUM_CONCEPTS = 1000000
B_DIM = 16384
J_DIM = 26
DIM = 32
NW = 32
BH = 128
BH_W = BH // NW
ROWS_W = BH_W * 128


N_BLOCKS = 3906
N_BLOCKS_W = 122
TAIL_I = N_BLOCKS * 256
BLK_W = 256 * DIM


def _detile(table_t, tail_lin):
    mesh = plsc.VectorSubcoreMesh(core_axis_name="c", subcore_axis_name="s")

    @functools.partial(
        pl.kernel,
        mesh=mesh,
        out_type=jax.ShapeDtypeStruct((NUM_CONCEPTS * DIM,), jnp.float32),
        scratch_types=[
            pltpu.VMEM((DIM, 256), jnp.float32),
            pltpu.VMEM((DIM, 256), jnp.float32),
            pltpu.VMEM((BLK_W,), jnp.float32),
            pltpu.VMEM((BLK_W,), jnp.float32),
            pltpu.SemaphoreType.DMA,
            pltpu.SemaphoreType.DMA,
        ],
        compiler_params=pltpu.CompilerParams(
            use_tc_tiling_on_sc=True, needs_layout_passes=False
        ),
    )
    def k(tab_hbm, tail_hbm, out_hbm, buf0, buf1, tb0, tb1, sem_in, sem_out):
        bufs = (buf0, buf1)
        tbs = (tb0, tb1)
        wid = lax.axis_index("s") * 2 + lax.axis_index("c")
        start = wid * N_BLOCKS_W
        iota = lax.iota(jnp.int32, 16)
        rots = [jnp.bitwise_and(iota + kk, 15) for kk in range(16)]

        def fire(c, par):
            pltpu.make_async_copy(
                tab_hbm.at[pl.ds(0, DIM), pl.ds(c * 256, 256)],
                bufs[par],
                sem_in,
            ).start()

        def wait_in(par):
            pltpu.make_async_copy(
                tab_hbm.at[pl.ds(0, DIM), pl.ds(0, 256)], bufs[par], sem_in
            ).wait()

        dstc = [rots[kk] * DIM + iota for kk in range(16)]

        def transpose(par):
            def tbody(m, carry):
                il0 = m * 16
                cols = [rots[kk] + il0 for kk in range(16)]
                for d0 in (0, 16):
                    base = il0 * DIM + d0
                    for h in range(2):
                        vs = []
                        for kk in range(8 * h, 8 * h + 8):
                            vs.append(
                                plsc.load_gather(
                                    bufs[par], [iota + d0, cols[kk]]
                                )
                            )
                        for i, kk in enumerate(range(8 * h, 8 * h + 8)):
                            plsc.store_scatter(tbs[par], [dstc[kk] + base], vs[i])
                return carry

            lax.fori_loop(0, 16, tbody, 0)

        def start_out(c, par):
            pltpu.make_async_copy(
                tbs[par], out_hbm.at[pl.ds(c * BLK_W, BLK_W)], sem_out
            ).start()

        def drain_out(par):
            pltpu.make_async_copy(
                tbs[par], out_hbm.at[pl.ds(0, BLK_W)], sem_out
            ).wait()

        fire(start, 0)
        fire(start + 1, 1)

        def body(tt, carry):
            for par in range(2):
                t = 2 * tt + par
                wait_in(par)

                @pl.when(tt > 0)
                def _():
                    drain_out(par)

                transpose(par)
                start_out(start + t, par)

                @pl.when(t + 2 < N_BLOCKS_W)
                def _():
                    fire(start + t + 2, par)

            return carry

        lax.fori_loop(0, N_BLOCKS_W // 2, body, 0)
        drain_out(0)
        drain_out(1)

        @pl.when(wid < 2)
        def _():
            c = NW * N_BLOCKS_W + wid
            pltpu.sync_copy(
                tab_hbm.at[pl.ds(0, DIM), pl.ds(c * 256, 256)], bufs[0]
            )
            transpose(0)
            pltpu.sync_copy(tbs[0], out_hbm.at[pl.ds(c * BLK_W, BLK_W)])

        @pl.when(wid == 2)
        def _():
            pltpu.sync_copy(tail_hbm, tb1.at[pl.ds(0, 64 * DIM)])
            pltpu.sync_copy(
                tb1.at[pl.ds(0, 64 * DIM)],
                out_hbm.at[pl.ds(TAIL_I * DIM, 64 * DIM)],
            )

    return k(table_t, tail_lin)


def _gather5d(idx_cols, table):
    mesh = plsc.VectorSubcoreMesh(core_axis_name="c", subcore_axis_name="s")

    @functools.partial(
        pl.kernel,
        mesh=mesh,
        out_type=jax.ShapeDtypeStruct((J_DIM, 4, BH * 8, 128), jnp.float32),
        scratch_types=[
            pltpu.VMEM((J_DIM, BH_W, 128), jnp.int32),
            pltpu.VMEM((2, ROWS_W, DIM), jnp.float32),
            pltpu.VMEM((2, 4, 40, 129), jnp.float32),
            pltpu.SemaphoreType.DMA,
            pltpu.SemaphoreType.DMA,
        ],
        compiler_params=pltpu.CompilerParams(
            use_tc_tiling_on_sc=False, needs_layout_passes=False
        ),
    )
    def k(idx_hbm, tab_hbm, out_hbm, idx_v, rows_v, tb_v, sem_in, sem_out):
        wid = lax.axis_index("s") * 2 + lax.axis_index("c")
        pltpu.sync_copy(idx_hbm.at[pl.ds(0, J_DIM), pl.ds(wid * BH_W, BH_W)], idx_v)
        iota = lax.iota(jnp.int32, 16)

        def fire(j, par):
            for b in range(BH_W):
                pltpu.make_async_copy(
                    tab_hbm.at[idx_v.at[j, b]],
                    rows_v.at[par, pl.ds(b * 128, 128)],
                    sem_in,
                ).start()

        def wait_in(par):
            for b in range(BH_W):
                pltpu.make_async_copy(
                    tab_hbm.at[idx_v.at[0, b]],
                    rows_v.at[par, pl.ds(b * 128, 128)],
                    sem_in,
                ).wait()

        dh_pat = lax.shift_right_logical(iota, 3)
        dl_pat = lax.bitwise_and(iota, 7)

        def transpose(par):
            parv = jnp.full((16,), par, jnp.int32)
            dh_hi = dh_pat + 2
            for b in range(BH_W):
                m_vec = dl_pat + (b * 8)

                def rbody(t, carry):
                    for u in range(8):
                        bl = t * 8 + u
                        r = b * 128 + bl
                        v1 = rows_v[par, r, pl.ds(0, 16)]
                        v2 = rows_v[par, r, pl.ds(16, 16)]
                        blv = jnp.full((16,), bl, jnp.int32)
                        plsc.store_scatter(tb_v, [parv, dh_pat, m_vec, blv], v1)
                        plsc.store_scatter(tb_v, [parv, dh_hi, m_vec, blv], v2)
                    return carry

                lax.fori_loop(0, 16, rbody, 0)

        def start_out(j, par):
            for dh in range(4):
                pltpu.make_async_copy(
                    tb_v.at[par, dh, pl.ds(0, 32), pl.ds(0, 128)],
                    out_hbm.at[j, dh, pl.ds(wid * 32, 32)],
                    sem_out,
                ).start()

        def drain_out(par):
            for dh in range(4):
                pltpu.make_async_copy(
                    tb_v.at[par, dh, pl.ds(0, 32), pl.ds(0, 128)],
                    out_hbm.at[0, dh, pl.ds(wid * 32, 32)],
                    sem_out,
                ).wait()

        fire(0, 0)

        def body(jj, carry):
            j0 = 2 * jj
            j1 = j0 + 1
            fire(j1, 1)
            wait_in(0)

            @pl.when(jj > 0)
            def _():
                drain_out(0)

            transpose(0)
            start_out(j0, 0)

            @pl.when(jj < J_DIM // 2 - 1)
            def _():
                fire(j0 + 2, 0)

            wait_in(1)

            @pl.when(jj > 0)
            def _():
                drain_out(1)

            transpose(1)
            start_out(j1, 1)
            return carry

        lax.fori_loop(0, J_DIM // 2, body, 0)
        drain_out(0)
        drain_out(1)

    return k(idx_cols, table)


def kernel(idx, table):
    idx_cols = idx.T.reshape(J_DIM, BH, 128).astype(jnp.int32)
    tail_lin = table[TAIL_I:].reshape(-1)
    tab_lin = _detile(table.T, tail_lin).reshape(NUM_CONCEPTS, DIM)
    out4 = _gather5d(idx_cols, tab_lin)
    out5d = out4.reshape(J_DIM, 4, BH, 8, 128)
    return out5d.transpose(2, 4, 0, 1, 3).reshape(B_DIM, J_DIM, DIM)
